# baseline (speedup 1.0000x reference)
import functools

import jax
import jax.numpy as jnp
from jax import lax
from jax.experimental import pallas as pl
from jax.experimental.pallas import tpu as pltpu
from jax.experimental.pallas import tpu_sc as plsc

BATCH = 8
VOCAB = 100000
D = 64
K = 10
L = 16

NC = 2
NS = 16

CHUNK_STRIDE = 25008
PROC = 24976
NVREG = PROC // L
GAP_OFFS = (24976, 49984, 74992)
GAP_LEN = 32

NEG_INF = float("-inf")


def _merge_top16(tv, ti, sv_desc, si_desc):
    take = sv_desc > tv
    nv = jnp.where(take, sv_desc, tv)
    ni = jnp.where(take, si_desc, ti)
    return plsc.sort_key_val(nv, ni, descending=False)


def _scores_tc(af2, wv):
    rows = af2.shape[0]
    rblk = 8000
    grid = rows // rblk

    def body(x_ref, w_ref, o_ref):
        o_ref[...] = jax.lax.dot_general(
            x_ref[...],
            w_ref[...],
            (((1,), (0,)), ((), ())),
            preferred_element_type=jnp.float32,
        )

    out = pl.pallas_call(
        body,
        grid=(grid,),
        in_specs=[
            pl.BlockSpec((rblk, D), lambda i: (i, 0)),
            pl.BlockSpec((D, 1), lambda i: (0, 0)),
        ],
        out_specs=pl.BlockSpec((rblk, 1), lambda i: (i, 0)),
        out_shape=jax.ShapeDtypeStruct((rows, 1), jnp.float32),
    )(af2, wv)
    return out.reshape(rows)


def _topk_gather_sc(af2, scores):
    mesh = plsc.VectorSubcoreMesh(
        core_axis_name="c", subcore_axis_name="s", num_cores=NC, num_subcores=NS
    )

    @functools.partial(
        pl.kernel,
        out_type=jax.ShapeDtypeStruct((BATCH, K, D), jnp.float32),
        mesh=mesh,
        compiler_params=pltpu.CompilerParams(
            needs_layout_passes=False, use_tc_tiling_on_sc=False
        ),
        scratch_types=[
            pltpu.VMEM((PROC,), jnp.float32),
            pltpu.VMEM((6 * L,), jnp.float32),
            pltpu.VMEM((L,), jnp.float32),
            pltpu.VMEM((L,), jnp.int32),
            pltpu.VMEM((4, L), jnp.float32),
            pltpu.VMEM((4, L), jnp.int32),
            pltpu.VMEM((L,), jnp.int32),
            pltpu.VMEM((L, D), jnp.float32),
            pltpu.VMEM_SHARED((NS, L), jnp.float32),
            pltpu.VMEM_SHARED((NS, L), jnp.int32),
            pltpu.SemaphoreType.DMA,
        ],
    )
    def k(af_hbm, sc_hbm, out_hbm, buf, gapbuf, pubv, pubi, cv, ci, idxv,
          rows_v, shv, shi, sem):
        core = lax.axis_index("c")
        sub = lax.axis_index("s")
        wid = core * NS + sub
        b = wid // 4
        c = wid % 4
        boff = b * VOCAB
        start = boff + c * CHUNK_STRIDE

        pltpu.sync_copy(sc_hbm.at[pl.ds(start, PROC)], buf)

        lane = lax.iota(jnp.int32, L)
        tv0 = jnp.full((L,), NEG_INF, jnp.float32)
        ti0 = jnp.zeros((L,), jnp.int32)

        def scan_body(i, carry):
            tv, ti, thr = carry
            v = buf[pl.ds(i * L, L)]

            def do_merge(_):
                gidx = start + i * L + lane
                sv, si = plsc.sort_key_val(v, gidx, descending=True)
                tv2, ti2 = _merge_top16(tv, ti, sv, si)
                return tv2, ti2, jnp.min(tv2)

            return lax.cond(
                jnp.any(v > thr), do_merge, lambda _: (tv, ti, thr), None
            )

        tv, ti, _ = lax.fori_loop(
            0, NVREG, scan_body, (tv0, ti0, jnp.float32(NEG_INF))
        )
        pubv[...] = tv
        pubi[...] = ti

        @pl.when(c == 0)
        def _():
            for g, off in enumerate(GAP_OFFS):
                pltpu.sync_copy(
                    sc_hbm.at[pl.ds(boff + off, GAP_LEN)],
                    gapbuf.at[pl.ds(g * GAP_LEN, GAP_LEN)],
                )
            gv = pubv[...]
            gi = pubi[...]
            for j in range(6):
                v = gapbuf[pl.ds(j * L, L)]
                gidx = boff + GAP_OFFS[j // 2] + (j % 2) * L + lane
                sv, si = plsc.sort_key_val(v, gidx, descending=True)
                gv, gi = _merge_top16(gv, gi, sv, si)
            pubv[...] = gv
            pubi[...] = gi

        pltpu.sync_copy(pubv, shv.at[sub])
        pltpu.sync_copy(pubi, shi.at[sub])
        plsc.subcore_barrier()

        @pl.when(c == 0)
        def _():
            pltpu.sync_copy(shv.at[pl.ds(sub, 4)], cv)
            pltpu.sync_copy(shi.at[pl.ds(sub, 4)], ci)
            mv = cv[0]
            mi = ci[0]
            for j in range(1, 4):
                rv = lax.rev(cv[j], (0,))
                ri = lax.rev(ci[j], (0,))
                mv, mi = _merge_top16(mv, mi, rv, ri)
            idxv[...] = lax.rev(mi, (0,))
            pltpu.async_copy(af_hbm.at[idxv], rows_v, sem).wait()
            pltpu.sync_copy(rows_v.at[pl.ds(0, K)], out_hbm.at[b])

    return k(af2, scores)


def kernel(activation_field, W, b):
    af2 = activation_field.reshape(BATCH * VOCAB, D)
    scores = _scores_tc(af2, W)
    return _topk_gather_sc(af2, scores)


# speedup vs baseline: 2.4821x; 2.4821x over previous
import functools

import jax
import jax.numpy as jnp
from jax import lax
from jax.experimental import pallas as pl
from jax.experimental.pallas import tpu as pltpu
from jax.experimental.pallas import tpu_sc as plsc

BATCH = 8
VOCAB = 100000
D = 64
K = 10
L = 16

NC = 2
NS = 16

CHUNK_STRIDE = 25008
PROC = 24976
UNROLL = 7
NGROUP = 223
GAP_OFFS = (24976, 49984, 74992)
GAP_LEN = 32

RBLK = 8192
NBLK = 98
SROWS = NBLK * 64

NEG_INF = float("-inf")


def _merge_top16(tv, ti, sv_desc, si_desc):
    take = sv_desc > tv
    nv = jnp.where(take, sv_desc, tv)
    ni = jnp.where(take, si_desc, ti)
    return plsc.sort_key_val(nv, ni, descending=False)


def _scores_tc(af2, W):
    def body(x_ref, w_ref, o_ref):
        y = lax.dot_general(
            x_ref[...],
            w_ref[...],
            (((1,), (0,)), ((), ())),
            preferred_element_type=jnp.float32,
        )
        o_ref[...] = y.reshape(64, 128)

    out = pl.pallas_call(
        body,
        grid=(NBLK,),
        in_specs=[
            pl.BlockSpec((RBLK, D), lambda i: (i, 0)),
            pl.BlockSpec((D, 1), lambda i: (0, 0)),
        ],
        out_specs=pl.BlockSpec((64, 128), lambda i: (i, 0)),
        out_shape=jax.ShapeDtypeStruct((SROWS, 128), jnp.float32),
    )(af2, W)
    return out.reshape(SROWS * 128)


def _topk_sc(scores):
    mesh = plsc.VectorSubcoreMesh(
        core_axis_name="c", subcore_axis_name="s", num_cores=NC, num_subcores=NS
    )

    @functools.partial(
        pl.kernel,
        out_type=jax.ShapeDtypeStruct((BATCH * L,), jnp.int32),
        mesh=mesh,
        compiler_params=pltpu.CompilerParams(
            needs_layout_passes=False, use_tc_tiling_on_sc=False
        ),
        scratch_types=[
            pltpu.VMEM((PROC,), jnp.float32),
            pltpu.VMEM((6 * L,), jnp.float32),
            pltpu.VMEM((L,), jnp.float32),
            pltpu.VMEM((L,), jnp.int32),
            pltpu.VMEM((4, L), jnp.float32),
            pltpu.VMEM((4, L), jnp.int32),
            pltpu.VMEM((L,), jnp.int32),
            pltpu.VMEM_SHARED((NS, L), jnp.float32),
            pltpu.VMEM_SHARED((NS, L), jnp.int32),
        ],
    )
    def k(sc_hbm, out_hbm, buf, gapbuf, pubv, pubi, cv, ci, outi, shv, shi):
        core = lax.axis_index("c")
        sub = lax.axis_index("s")
        wid = core * NS + sub
        b = wid // 4
        c = wid % 4
        boff = b * VOCAB
        start = boff + c * CHUNK_STRIDE

        pltpu.sync_copy(sc_hbm.at[pl.ds(start, PROC)], buf)

        lane = lax.iota(jnp.int32, L)
        tv0 = jnp.full((L,), NEG_INF, jnp.float32)
        ti0 = jnp.zeros((L,), jnp.int32)

        def group_body(g, carry):
            base = g * (UNROLL * L)
            vs = [buf[pl.ds(base + j * L, L)] for j in range(UNROLL)]
            m = vs[0]
            for j in range(1, UNROLL):
                m = jnp.maximum(m, vs[j])

            def hit(carry):
                tv, ti, thr = carry
                for j in range(UNROLL):
                    v = vs[j]

                    def do_merge(cr, v=v, j=j):
                        tv, ti, _ = cr
                        gidx = start + base + j * L + lane
                        sv, si = plsc.sort_key_val(v, gidx, descending=True)
                        tv2, ti2 = _merge_top16(tv, ti, sv, si)
                        return tv2, ti2, jnp.min(tv2)

                    tv, ti, thr = lax.cond(
                        jnp.any(v > thr), do_merge, lambda cr: cr, (tv, ti, thr)
                    )
                return tv, ti, thr

            return lax.cond(jnp.any(m > carry[2]), hit, lambda cr: cr, carry)

        tv, ti, _ = lax.fori_loop(
            0, NGROUP, group_body, (tv0, ti0, jnp.float32(NEG_INF))
        )
        pubv[...] = tv
        pubi[...] = ti

        @pl.when(c == 0)
        def _():
            for g, off in enumerate(GAP_OFFS):
                pltpu.sync_copy(
                    sc_hbm.at[pl.ds(boff + off, GAP_LEN)],
                    gapbuf.at[pl.ds(g * GAP_LEN, GAP_LEN)],
                )
            gv = pubv[...]
            gi = pubi[...]
            for j in range(6):
                v = gapbuf[pl.ds(j * L, L)]
                gidx = boff + GAP_OFFS[j // 2] + (j % 2) * L + lane
                sv, si = plsc.sort_key_val(v, gidx, descending=True)
                gv, gi = _merge_top16(gv, gi, sv, si)
            pubv[...] = gv
            pubi[...] = gi

        pltpu.sync_copy(pubv, shv.at[sub])
        pltpu.sync_copy(pubi, shi.at[sub])
        plsc.subcore_barrier()

        @pl.when(c == 0)
        def _():
            pltpu.sync_copy(shv.at[pl.ds(sub, 4)], cv)
            pltpu.sync_copy(shi.at[pl.ds(sub, 4)], ci)
            mv = cv[0]
            mi = ci[0]
            for j in range(1, 4):
                rv = lax.rev(cv[j], (0,))
                ri = lax.rev(ci[j], (0,))
                mv, mi = _merge_top16(mv, mi, rv, ri)
            outi[...] = lax.rev(mi, (0,))
            pltpu.sync_copy(outi, out_hbm.at[pl.ds(b * L, L)])

    return k(scores)


def _gather_tc(af2, idx):
    def body(idx_sref, *refs):
        o_ref = refs[L]
        bb = pl.program_id(0)
        rows = []
        for j in range(L):
            h = idx_sref[bb * L + j] % 8
            x = refs[j][...]
            acc = jnp.where(h == 0, x[0:1, :], 0.0)
            for hh in range(1, 8):
                acc = acc + jnp.where(h == hh, x[hh : hh + 1, :], 0.0)
            rows.append(acc)
        o_ref[...] = jnp.concatenate(rows, axis=0)[None]

    def mk_spec(j):
        return pl.BlockSpec(
            (8, D), lambda bb, idx_ref, j=j: (idx_ref[bb * L + j] // 8, 0)
        )

    out = pl.pallas_call(
        body,
        grid_spec=pltpu.PrefetchScalarGridSpec(
            num_scalar_prefetch=1,
            grid=(BATCH,),
            in_specs=[mk_spec(j) for j in range(L)],
            out_specs=pl.BlockSpec((1, L, D), lambda bb, idx_ref: (bb, 0, 0)),
        ),
        out_shape=jax.ShapeDtypeStruct((BATCH, L, D), jnp.float32),
    )(idx, *([af2] * L))
    return out


def kernel(activation_field, W, b):
    af2 = activation_field.reshape(BATCH * VOCAB, D)
    scores = _scores_tc(af2, W)
    idx = _topk_sc(scores)
    gathered = _gather_tc(af2, idx)
    return gathered[:, :K, :]
